# baseline (speedup 1.0000x reference)
import jax
import jax.numpy as jnp
from jax import lax
from jax.experimental import pallas as pl
from jax.experimental.pallas import tpu as pltpu
from jax.experimental.pallas import tpu_sc as plsc

B = 4
T = 2048
D = 128
NC = 2
NS = 16
NW = NC * NS
PPW = T // NW
LANES = 16


def _emb_body(x_hbm, tok_hbm, pos_hbm, out_hbm, idx_v, tok_v, pos_v, gsem, psem, osem, isem):
    wid = lax.axis_index("s") * NC + lax.axis_index("c")
    off = wid * PPW

    pcp = pltpu.async_copy(pos_hbm.at[pl.ds(off, PPW)], pos_v, psem)
    icps = [
        pltpu.async_copy(x_hbm.at[b, pl.ds(off, PPW)], idx_v.at[b], isem)
        for b in range(B)
    ]
    gcps = []
    for b in range(B):
        icps[b].wait()
        gcps.append(pltpu.async_copy(tok_hbm.at[idx_v.at[b]], tok_v.at[b], gsem))
    pcp.wait()

    for cp in gcps:
        cp.wait()

    @plsc.parallel_loop(0, B * PPW * (D // LANES), 1, unroll=1)
    def lane_add(i):
        r = lax.div(i, D // LANES)
        sl = pl.ds(lax.rem(i, D // LANES) * LANES, LANES)
        plsc.addupdate(
            tok_v.at[lax.div(r, PPW), lax.rem(r, PPW), sl],
            pos_v[lax.rem(r, PPW), sl],
        )

    pltpu.sync_copy(tok_v, out_hbm.at[:, pl.ds(off, PPW), :])


@jax.jit
def kernel(x, tok_table, pos_table):
    run = pl.kernel(
        _emb_body,
        out_type=jax.ShapeDtypeStruct((B, T, D), jnp.float32),
        mesh=plsc.VectorSubcoreMesh(core_axis_name="c", subcore_axis_name="s"),
        scratch_types=[
            pltpu.VMEM((B, PPW), jnp.int32),
            pltpu.VMEM((B, PPW, D), jnp.float32),
            pltpu.VMEM((PPW, D), jnp.float32),
            pltpu.SemaphoreType.DMA,
            pltpu.SemaphoreType.DMA,
            pltpu.SemaphoreType.DMA,
            pltpu.SemaphoreType.DMA,
        ],
    )
    return run(x.astype(jnp.int32), tok_table, pos_table)


# speedup vs baseline: 1.3367x; 1.3367x over previous
import jax
import jax.numpy as jnp
from jax import lax
from jax.experimental import pallas as pl
from jax.experimental.pallas import tpu as pltpu
from jax.experimental.pallas import tpu_sc as plsc

B = 4
T = 2048
D = 128
NC = 2
NS = 16
NW = NC * NS
PPW = T // NW
LANES = 16


def _emb_body(x_hbm, tok_hbm, pos_hbm, out_hbm, idx_v, tok_v, pos_v, gsem, psem, osem, isem):
    wid = lax.axis_index("s") * NC + lax.axis_index("c")
    off = wid * PPW

    pcp = pltpu.async_copy(pos_hbm.at[pl.ds(off, PPW)], pos_v, psem)
    icps = [
        pltpu.async_copy(x_hbm.at[b, pl.ds(off, PPW)], idx_v.at[b], isem)
        for b in range(B)
    ]
    gcps = []
    for b in range(B):
        icps[b].wait()
        gcps.append(pltpu.async_copy(tok_hbm.at[idx_v.at[b]], tok_v.at[b], gsem))
    pcp.wait()

    ocps = []
    for b in range(B):
        gcps[b].wait()

        @plsc.parallel_loop(0, PPW, 1, unroll=1)
        def row_add(r):
            for c in range(D // LANES):
                sl = pl.ds(c * LANES, LANES)
                plsc.addupdate(tok_v.at[b, r, sl], pos_v[r, sl])

        ocps.append(
            pltpu.async_copy(tok_v.at[b], out_hbm.at[b, pl.ds(off, PPW), :], osem)
        )
    for cp in ocps:
        cp.wait()


@jax.jit
def kernel(x, tok_table, pos_table):
    run = pl.kernel(
        _emb_body,
        out_type=jax.ShapeDtypeStruct((B, T, D), jnp.float32),
        mesh=plsc.VectorSubcoreMesh(core_axis_name="c", subcore_axis_name="s"),
        scratch_types=[
            pltpu.VMEM((B, PPW), jnp.int32),
            pltpu.VMEM((B, PPW, D), jnp.float32),
            pltpu.VMEM((PPW, D), jnp.float32),
            pltpu.SemaphoreType.DMA,
            pltpu.SemaphoreType.DMA,
            pltpu.SemaphoreType.DMA,
            pltpu.SemaphoreType.DMA,
        ],
    )
    return run(x.astype(jnp.int32), tok_table, pos_table)
